# baseline (speedup 1.0000x reference)
import functools

import jax
import jax.numpy as jnp
from jax.experimental import pallas as pl

_N = 512
_H = 200
_HP = 256


def _pad_gate_mat(w, in_p):
    parts = []
    for g in range(3):
        wg = w[g * _H:(g + 1) * _H, :].T
        wg = jnp.pad(wg, ((0, in_p - wg.shape[0]), (0, _HP - _H)))
        parts.append(wg)
    return jnp.concatenate(parts, axis=1)


def _pad_gate_bias(b):
    parts = []
    for g in range(3):
        parts.append(jnp.pad(b[g * _H:(g + 1) * _H], (0, _HP - _H)))
    return jnp.concatenate(parts)


def _dot(a, b):
    return jax.lax.dot_general(a, b, (((1,), (0,)), ((), ())),
                               preferred_element_type=jnp.float32)


def _dot_t(a, b):
    return jax.lax.dot_general(a, b, (((0,), (0,)), ((), ())),
                               preferred_element_type=jnp.float32)


def _fused_body(adj_ref, gin_ref, ke_ref, wi_ref, wh_ref, bi_ref, bh_ref,
                wm_ref, bm_ref, out_ref, *, thr, pair):
    f32 = jnp.float32
    bf16 = jnp.bfloat16

    def gru(idx, xb, h, hb):
        gx = _dot(xb, wi_ref[idx]) + bi_ref[idx:idx + 1, :]
        if hb is None:
            gh = jnp.broadcast_to(bh_ref[idx:idx + 1, :], gx.shape)
        else:
            gh = _dot(hb, wh_ref[idx]) + bh_ref[idx:idx + 1, :]
        r = jax.nn.sigmoid(gx[:, 0:_HP] + gh[:, 0:_HP])
        z = jax.nn.sigmoid(gx[:, _HP:2 * _HP] + gh[:, _HP:2 * _HP])
        n = jnp.tanh(gx[:, 2 * _HP:] + r * gh[:, 2 * _HP:])
        if h is None:
            return (1.0 - z) * n
        return (1.0 - z) * n + z * h

    def graph_chain(g):
        a = (adj_ref[g] < thr).astype(f32)
        deg_in = jnp.maximum(jnp.sum(a, axis=1, keepdims=True), 1.0)
        deg_out = jnp.maximum(jnp.sum(a, axis=0, keepdims=True), 1.0)
        ar = (a / deg_in).astype(bf16)
        ac = (a / deg_out).astype(bf16)

        h = gru(0, _dot(ar, gin_ref[g]).astype(bf16), None, None)
        vo0 = h[0:3, :]
        hb = h.astype(bf16)
        h = gru(1, _dot_t(ac, hb).astype(bf16), h, hb)
        hb = h.astype(bf16)
        h = gru(2, _dot(ar, hb).astype(bf16), h, hb)
        vo1 = h[0:3, :]
        hb = h.astype(bf16)
        h = gru(3, _dot_t(ac, hb).astype(bf16), h, hb)
        hb = h.astype(bf16)
        m3 = _dot(ar[0:3, :], hb).astype(bf16)
        vo2 = gru(4, m3, h[0:3, :], hb[0:3, :])

        hv = gru(5, vo0.astype(bf16), None, None)
        hv = gru(5, vo1.astype(bf16), hv, hv.astype(bf16))
        hv = gru(5, vo2.astype(bf16), hv, hv.astype(bf16))

        enc = (_dot(hv[0:1, :], wm_ref[0:_HP, :]) +
               _dot(hv[1:2, :], wm_ref[_HP:2 * _HP, :]) +
               _dot(hv[2:3, :], wm_ref[2 * _HP:, :]) + bm_ref[...])
        d = enc - ke_ref[g]
        return jnp.sum(d * d)

    loss = graph_chain(0)
    for g in range(1, pair):
        loss = loss + graph_chain(g)
    loss = loss.reshape(1, 1)

    b = pl.program_id(0)

    @pl.when(b == 0)
    def _():
        out_ref[...] = loss

    @pl.when(b != 0)
    def _():
        out_ref[...] += loss


def kernel(g_in, g_adj, batch_size, kernel_embeddings, reg_solutions, params):
    del reg_solutions
    b, n, vt = g_in.shape
    thr = 16.0 / n
    pair = 2 if b % 2 == 0 else 1

    grus = [params["fw"][0], params["bw"][0], params["fw"][1],
            params["bw"][1], params["fw"][2], params["var"]]
    wi_all = jnp.stack([_pad_gate_mat(p["Wi"], _HP) for p in grus]
                       ).astype(jnp.bfloat16)
    wh_all = jnp.stack([_pad_gate_mat(p["Wh"], _HP) for p in grus]
                       ).astype(jnp.bfloat16)
    bi_all = jnp.stack([_pad_gate_bias(p["bi"]) for p in grus])
    bh_all = jnp.stack([_pad_gate_bias(p["bh"]) for p in grus])
    z = params["Wm"].shape[0]
    wm = params["Wm"].reshape(z, 3, _H).transpose(1, 2, 0)
    wm = jnp.pad(wm, ((0, 0), (0, _HP - _H), (0, 0))).reshape(3 * _HP, z)
    bm = params["bm"].reshape(1, z)

    full = lambda shape: pl.BlockSpec(shape, lambda i: (0,) * len(shape))

    out = pl.pallas_call(
        functools.partial(_fused_body, thr=thr, pair=pair),
        grid=(b // pair,),
        in_specs=[
            pl.BlockSpec((pair, n, n), lambda i: (i, 0, 0)),
            pl.BlockSpec((pair, n, vt), lambda i: (i, 0, 0)),
            pl.BlockSpec((pair, 1, z), lambda i: (i, 0, 0)),
            full(wi_all.shape),
            full(wh_all.shape),
            full(bi_all.shape),
            full(bh_all.shape),
            full(wm.shape),
            full(bm.shape),
        ],
        out_specs=pl.BlockSpec((1, 1), lambda i: (0, 0)),
        out_shape=jax.ShapeDtypeStruct((1, 1), jnp.float32),
    )(g_adj, g_in.astype(jnp.bfloat16), kernel_embeddings.reshape(b, 1, z),
      wi_all, wh_all, bi_all, bh_all, wm, bm)
    return out[0, 0]


# speedup vs baseline: 1.0618x; 1.0618x over previous
import functools

import jax
import jax.numpy as jnp
from jax.experimental import pallas as pl

_N = 512
_H = 200
_HP = 256


def _pad_gate_mat(w, in_p):
    parts = []
    for g in range(3):
        wg = w[g * _H:(g + 1) * _H, :].T
        wg = jnp.pad(wg, ((0, in_p - wg.shape[0]), (0, _HP - _H)))
        parts.append(wg)
    return jnp.concatenate(parts, axis=1)


def _pad_gate_bias(b):
    parts = []
    for g in range(3):
        parts.append(jnp.pad(b[g * _H:(g + 1) * _H], (0, _HP - _H)))
    return jnp.concatenate(parts)


def _dot(a, b):
    return jax.lax.dot_general(a, b, (((1,), (0,)), ((), ())),
                               preferred_element_type=jnp.float32)


def _dot_t(a, b):
    return jax.lax.dot_general(a, b, (((0,), (0,)), ((), ())),
                               preferred_element_type=jnp.float32)


def _fused_body(adj_ref, gin_ref, ke_ref, wi_ref, wh_ref, bi_ref, bh_ref,
                wm_ref, bm_ref, out_ref, *, thr, pair):
    f32 = jnp.float32
    bf16 = jnp.bfloat16

    def gru(idx, xb, h, hb):
        gx = _dot(xb, wi_ref[idx]) + bi_ref[idx:idx + 1, :]
        if hb is None:
            gh = jnp.broadcast_to(bh_ref[idx:idx + 1, :], gx.shape)
        else:
            gh = _dot(hb, wh_ref[idx]) + bh_ref[idx:idx + 1, :]
        r = jax.nn.sigmoid(gx[:, 0:_HP] + gh[:, 0:_HP])
        z = jax.nn.sigmoid(gx[:, _HP:2 * _HP] + gh[:, _HP:2 * _HP])
        n = jnp.tanh(gx[:, 2 * _HP:] + r * gh[:, 2 * _HP:])
        if h is None:
            return (1.0 - z) * n
        return (1.0 - z) * n + z * h

    def graph_chain(g):
        a = (adj_ref[g] < thr).astype(f32)
        deg_in = jnp.maximum(jnp.sum(a, axis=1, keepdims=True), 1.0)
        deg_out = jnp.maximum(jnp.sum(a, axis=0, keepdims=True), 1.0)
        ar = a / deg_in
        ac = a / deg_out

        h = gru(0, _dot(ar, gin_ref[g]), None, None)
        vo0 = h[0:3, :]
        h = gru(1, _dot_t(ac, h), h, h)
        h = gru(2, _dot(ar, h), h, h)
        vo1 = h[0:3, :]
        h = gru(3, _dot_t(ac, h), h, h)
        m3 = _dot(ar[0:3, :], h)
        vo2 = gru(4, m3, h[0:3, :], h[0:3, :])

        hv = gru(5, vo0, None, None)
        hv = gru(5, vo1, hv, hv)
        hv = gru(5, vo2, hv, hv)

        enc = (_dot(hv[0:1, :], wm_ref[0:_HP, :]) +
               _dot(hv[1:2, :], wm_ref[_HP:2 * _HP, :]) +
               _dot(hv[2:3, :], wm_ref[2 * _HP:, :]) + bm_ref[...])
        d = enc - ke_ref[g]
        return jnp.sum(d * d)

    loss = graph_chain(0)
    for g in range(1, pair):
        loss = loss + graph_chain(g)
    loss = loss.reshape(1, 1)

    b = pl.program_id(0)

    @pl.when(b == 0)
    def _():
        out_ref[...] = loss

    @pl.when(b != 0)
    def _():
        out_ref[...] += loss


def kernel(g_in, g_adj, batch_size, kernel_embeddings, reg_solutions, params):
    del reg_solutions
    b, n, vt = g_in.shape
    thr = 16.0 / n
    pair = 2 if b % 2 == 0 else 1

    grus = [params["fw"][0], params["bw"][0], params["fw"][1],
            params["bw"][1], params["fw"][2], params["var"]]
    wi_all = jnp.stack([_pad_gate_mat(p["Wi"], _HP) for p in grus])
    wh_all = jnp.stack([_pad_gate_mat(p["Wh"], _HP) for p in grus])
    bi_all = jnp.stack([_pad_gate_bias(p["bi"]) for p in grus])
    bh_all = jnp.stack([_pad_gate_bias(p["bh"]) for p in grus])
    z = params["Wm"].shape[0]
    wm = params["Wm"].reshape(z, 3, _H).transpose(1, 2, 0)
    wm = jnp.pad(wm, ((0, 0), (0, _HP - _H), (0, 0))).reshape(3 * _HP, z)
    bm = params["bm"].reshape(1, z)

    full = lambda shape: pl.BlockSpec(shape, lambda i: (0,) * len(shape))

    out = pl.pallas_call(
        functools.partial(_fused_body, thr=thr, pair=pair),
        grid=(b // pair,),
        in_specs=[
            pl.BlockSpec((pair, n, n), lambda i: (i, 0, 0)),
            pl.BlockSpec((pair, n, vt), lambda i: (i, 0, 0)),
            pl.BlockSpec((pair, 1, z), lambda i: (i, 0, 0)),
            full(wi_all.shape),
            full(wh_all.shape),
            full(bi_all.shape),
            full(bh_all.shape),
            full(wm.shape),
            full(bm.shape),
        ],
        out_specs=pl.BlockSpec((1, 1), lambda i: (0, 0)),
        out_shape=jax.ShapeDtypeStruct((1, 1), jnp.float32),
    )(g_adj, g_in, kernel_embeddings.reshape(b, 1, z),
      wi_all, wh_all, bi_all, bh_all, wm, bm)
    return out[0, 0]
